# Initial kernel scaffold; baseline (speedup 1.0000x reference)
import jax
import jax.numpy as jnp
from jax.experimental import pallas as pl


def kernel(memory, unique_node_ids, unique_messages, timestamps, neighbors, edge_times, W_s):
    raise NotImplementedError("write your pallas kernel here")


# speedup vs baseline: 1.5174x; 1.5174x over previous
import functools

import jax
import jax.numpy as jnp
from jax import lax
from jax.experimental import pallas as pl
from jax.experimental.pallas import tpu as pltpu
from jax.experimental.pallas import tpu_sc as plsc

_M = 100000
_D = 64
_B = 4096
_NN = 20
_ALPHA = 1.0 / 100.0
_TAU = 200.0

_NCORE = 2
_NSUB = 16
_NW = _NCORE * _NSUB
_JPW = (_B * _NN) // _NW
_SPW = _B // _NW
_GC = 128
_NCH = _JPW // _GC

_mesh = plsc.VectorSubcoreMesh(core_axis_name="c", subcore_axis_name="s",
                               num_cores=_NCORE, num_subcores=_NSUB)

_sc_params = pltpu.CompilerParams(use_tc_tiling_on_sc=False)

_sc_scratch = [
    pltpu.VMEM((_GC,), jnp.int32),
    pltpu.VMEM((_GC,), jnp.int32),
    pltpu.VMEM((_GC, _D), jnp.float32),
    pltpu.VMEM((_GC, _D), jnp.float32),
] + [pltpu.SemaphoreType.DMA] * 6


def _worker_id():
    return lax.axis_index("c") * _NSUB + lax.axis_index("s")


@functools.partial(
    pl.kernel,
    out_type=(
        jax.ShapeDtypeStruct((_B * _NN, _D), jnp.float32),
        jax.ShapeDtypeStruct((_B, _D), jnp.float32),
    ),
    mesh=_mesh,
    scratch_types=_sc_scratch,
    compiler_params=_sc_params,
)
def _sc_gather(mem_hbm, flat_hbm, uid_hbm, nc_hbm, s_hbm,
               i0, i1, r0, r1, si0, si1, sg0, sg1, ss0, ss1):
    wid = _worker_id()
    jb = wid * _JPW
    sb = wid * _SPW
    ibufs, rbufs = (i0, i1), (r0, r1)
    isems, gsems, ssems = (si0, si1), (sg0, sg1), (ss0, ss1)

    pltpu.sync_copy(uid_hbm.at[pl.ds(sb, _SPW)], i0)
    pltpu.async_copy(mem_hbm.at[i0], r0, sg0).wait()
    pltpu.sync_copy(r0, s_hbm.at[pl.ds(sb, _SPW)])

    ih = [None] * _NCH
    gh = [None] * _NCH
    sh = [None] * _NCH

    def issue_i(c):
        b = c % 2
        ih[c] = pltpu.async_copy(
            flat_hbm.at[pl.ds(jb + c * _GC, _GC)], ibufs[b], isems[b])

    def issue_g(c):
        b = c % 2
        gh[c] = pltpu.async_copy(mem_hbm.at[ibufs[b]], rbufs[b], gsems[b])

    def issue_s(c):
        b = c % 2
        sh[c] = pltpu.async_copy(
            rbufs[b], nc_hbm.at[pl.ds(jb + c * _GC, _GC)], ssems[b])

    issue_i(0)
    issue_i(1)
    ih[0].wait()
    issue_g(0)
    ih[1].wait()
    issue_g(1)
    for c in range(_NCH):
        gh[c].wait()
        issue_s(c)
        sh[c].wait()
        if c + 2 < _NCH:
            issue_i(c + 2)
            ih[c + 2].wait()
            issue_g(c + 2)


@functools.partial(
    pl.kernel,
    out_type=(),
    mesh=_mesh,
    scratch_types=_sc_scratch,
    compiler_params=_sc_params,
)
def _sc_scatter(o_hbm, cv_hbm, hv_hbm, f0_hbm, f1_hbm,
                i0, i1, r0, r1, si0, si1, sr0, sr1, sw0, sw1):
    wid = _worker_id()
    jb = wid * _JPW
    ibufs, rbufs = (i0, i1), (r0, r1)
    isems, rsems, wsems = (si0, si1), (sr0, sr1), (sw0, sw1)

    for src_hbm, f_hbm in ((cv_hbm, f0_hbm), (hv_hbm, f1_hbm)):
        ih = [None] * _NCH
        rh = [None] * _NCH
        wh = [None] * _NCH

        def issue_load(c, src_hbm=src_hbm, f_hbm=f_hbm, ih=ih, rh=rh):
            b = c % 2
            ih[c] = pltpu.async_copy(
                f_hbm.at[pl.ds(jb + c * _GC, _GC)], ibufs[b], isems[b])
            rh[c] = pltpu.async_copy(
                src_hbm.at[pl.ds(jb + c * _GC, _GC)], rbufs[b], rsems[b])

        issue_load(0)
        issue_load(1)
        for c in range(_NCH):
            b = c % 2
            ih[c].wait()
            rh[c].wait()
            wh[c] = pltpu.async_copy(rbufs[b], o_hbm.at[ibufs[b]], wsems[b])
            wh[c].wait()
            if c + 2 < _NCH:
                issue_load(c + 2)


def _copy_body(x_ref, o_ref):
    o_ref[...] = x_ref[...]


def _tc_copy(mem2):
    rows = 2 * _M
    blk = 5000
    return pl.pallas_call(
        _copy_body,
        out_shape=jax.ShapeDtypeStruct((rows, _D), jnp.float32),
        grid=(rows // blk,),
        in_specs=[pl.BlockSpec((blk, _D), lambda i: (i, 0))],
        out_specs=pl.BlockSpec((blk, _D), lambda i: (i, 0)),
    )(mem2)


def _proj_body(m_ref, w_ref, o_ref):
    o_ref[...] = jnp.dot(m_ref[...], w_ref[...],
                         preferred_element_type=jnp.float32)


def _tc_project(msgs, w):
    return pl.pallas_call(
        _proj_body,
        out_shape=jax.ShapeDtypeStruct((_B, _D), jnp.float32),
    )(msgs, w)


_BB = 256


def _compute_body(nc_ref, s_ref, ts_ref, ets_ref, p_ref, cv_ref, hv_ref):
    nc = nc_ref[...]
    s = s_ref[...]
    logits = jnp.sum(nc * s, axis=2, keepdims=True)
    m = jnp.max(logits, axis=1, keepdims=True)
    e = jnp.exp(logits - m)
    att = e / jnp.sum(e, axis=1, keepdims=True)
    delta = ts_ref[...] - ets_ref[...]
    ok = jnp.logical_and(delta > 0.0, delta < _TAU)
    coef = jnp.where(ok, jnp.exp(-_ALPHA * delta) * att, 0.0)
    cv = nc + coef * p_ref[...]
    cv_ref[...] = cv
    hv_ref[...] = jnp.tanh(cv)


def _tc_compute(nc3, s3, ts3, ets3, p_rep):
    grid = _B // _BB
    sds = jax.ShapeDtypeStruct((_B, _NN, _D), jnp.float32)
    return pl.pallas_call(
        _compute_body,
        out_shape=(sds, sds),
        grid=(grid,),
        in_specs=[
            pl.BlockSpec((_BB, _NN, _D), lambda k: (k, 0, 0)),
            pl.BlockSpec((_BB, 1, _D), lambda k: (k, 0, 0)),
            pl.BlockSpec((_BB, 1, 1), lambda k: (k, 0, 0)),
            pl.BlockSpec((_BB, _NN, 1), lambda k: (k, 0, 0)),
            pl.BlockSpec((_BB, _NN, _D), lambda k: (lax.rem(k, 4), 0, 0)),
        ],
        out_specs=(
            pl.BlockSpec((_BB, _NN, _D), lambda k: (k, 0, 0)),
            pl.BlockSpec((_BB, _NN, _D), lambda k: (k, 0, 0)),
        ),
    )(nc3, s3, ts3, ets3, p_rep)


def kernel(memory, unique_node_ids, unique_messages, timestamps,
           neighbors, edge_times, W_s):
    mem2 = memory.reshape(2 * _M, _D)
    flat = neighbors.reshape(-1).astype(jnp.int32)
    flat1 = flat + jnp.int32(_M)
    uids = unique_node_ids.astype(jnp.int32)

    p = _tc_project(unique_messages, W_s)
    p_rep = jnp.tile(p, (5, 1)).reshape(_B // 4, _NN, _D)

    nc_flat, s_flat = _sc_gather(mem2, flat, uids)
    nc3 = nc_flat.reshape(_B, _NN, _D)
    s3 = s_flat.reshape(_B, 1, _D)
    ts3 = timestamps.reshape(_B, 1, 1)
    ets3 = edge_times.reshape(_B, _NN, 1)

    cv3, hv3 = _tc_compute(nc3, s3, ts3, ets3, p_rep)
    cv = cv3.reshape(_B * _NN, _D)
    hv = hv3.reshape(_B * _NN, _D)

    base = _tc_copy(mem2)
    out_ref = jax.new_ref(base)
    _sc_scatter(out_ref, cv, hv, flat, flat1)
    return out_ref[...].reshape(2, _M, _D)


# speedup vs baseline: 1.9048x; 1.2552x over previous
import functools

import jax
import jax.numpy as jnp
from jax import lax
from jax.experimental import pallas as pl
from jax.experimental.pallas import tpu as pltpu
from jax.experimental.pallas import tpu_sc as plsc

_M = 100000
_D = 64
_B = 4096
_NN = 20
_ALPHA = 1.0 / 100.0
_TAU = 200.0

_NCORE = 2
_NSUB = 16
_NW = _NCORE * _NSUB
_JPW = (_B * _NN) // _NW
_SPW = _B // _NW
_GC = 128
_NCH = _JPW // _GC

_mesh = plsc.VectorSubcoreMesh(core_axis_name="c", subcore_axis_name="s",
                               num_cores=_NCORE, num_subcores=_NSUB)

_sc_params = pltpu.CompilerParams(use_tc_tiling_on_sc=False)

_sc_scratch = [
    pltpu.VMEM((_GC,), jnp.int32),
    pltpu.VMEM((_GC,), jnp.int32),
    pltpu.VMEM((_GC, _D), jnp.float32),
    pltpu.VMEM((_GC, _D), jnp.float32),
] + [pltpu.SemaphoreType.DMA] * 6


def _worker_id():
    return lax.axis_index("c") * _NSUB + lax.axis_index("s")


@functools.partial(
    pl.kernel,
    out_type=(
        jax.ShapeDtypeStruct((_B * _NN, _D), jnp.float32),
        jax.ShapeDtypeStruct((_B, _D), jnp.float32),
    ),
    mesh=_mesh,
    scratch_types=_sc_scratch,
    compiler_params=_sc_params,
)
def _sc_gather(mem_hbm, flat_hbm, uid_hbm, nc_hbm, s_hbm,
               i0, i1, r0, r1, si0, si1, sg0, sg1, ss0, ss1):
    wid = _worker_id()
    jb = wid * _JPW
    sb = wid * _SPW
    ibufs, rbufs = (i0, i1), (r0, r1)
    isems, gsems, ssems = (si0, si1), (sg0, sg1), (ss0, ss1)

    pltpu.sync_copy(uid_hbm.at[pl.ds(sb, _SPW)], i0)
    pltpu.async_copy(mem_hbm.at[i0], r0, sg0).wait()
    pltpu.sync_copy(r0, s_hbm.at[pl.ds(sb, _SPW)])

    ih = [None] * _NCH
    gh = [None] * _NCH
    sh = [None] * _NCH

    def issue_i(c):
        b = c % 2
        ih[c] = pltpu.async_copy(
            flat_hbm.at[pl.ds(jb + c * _GC, _GC)], ibufs[b], isems[b])

    def issue_g(c):
        b = c % 2
        gh[c] = pltpu.async_copy(mem_hbm.at[ibufs[b]], rbufs[b], gsems[b])

    def issue_s(c):
        b = c % 2
        sh[c] = pltpu.async_copy(
            rbufs[b], nc_hbm.at[pl.ds(jb + c * _GC, _GC)], ssems[b])

    issue_i(0)
    issue_i(1)
    ih[0].wait()
    issue_g(0)
    ih[1].wait()
    issue_g(1)
    for c in range(_NCH):
        gh[c].wait()
        issue_s(c)
        sh[c].wait()
        if c + 2 < _NCH:
            issue_i(c + 2)
            ih[c + 2].wait()
            issue_g(c + 2)


@functools.partial(
    pl.kernel,
    out_type=(),
    mesh=_mesh,
    scratch_types=_sc_scratch,
    compiler_params=_sc_params,
)
def _sc_scatter(o_hbm, cv_hbm, hv_hbm, f0_hbm, f1_hbm,
                i0, i1, r0, r1, si0, si1, sr0, sr1, sw0, sw1):
    wid = _worker_id()
    jb = wid * _JPW
    ibufs, rbufs = (i0, i1), (r0, r1)
    isems, rsems, wsems = (si0, si1), (sr0, sr1), (sw0, sw1)

    for src_hbm, f_hbm in ((cv_hbm, f0_hbm), (hv_hbm, f1_hbm)):
        ih = [None] * _NCH
        rh = [None] * _NCH
        wh = [None] * _NCH

        def issue_load(c, src_hbm=src_hbm, f_hbm=f_hbm, ih=ih, rh=rh):
            b = c % 2
            ih[c] = pltpu.async_copy(
                f_hbm.at[pl.ds(jb + c * _GC, _GC)], ibufs[b], isems[b])
            rh[c] = pltpu.async_copy(
                src_hbm.at[pl.ds(jb + c * _GC, _GC)], rbufs[b], rsems[b])

        issue_load(0)
        issue_load(1)
        for c in range(_NCH):
            b = c % 2
            ih[c].wait()
            rh[c].wait()
            wh[c] = pltpu.async_copy(rbufs[b], o_hbm.at[ibufs[b]], wsems[b])
            wh[c].wait()
            if c + 2 < _NCH:
                issue_load(c + 2)


def _copy_body(x_ref, o_ref):
    o_ref[...] = x_ref[...]


def _tc_copy(mem2):
    rows = 2 * _M
    blk = 5000
    return pl.pallas_call(
        _copy_body,
        out_shape=jax.ShapeDtypeStruct((rows, _D), jnp.float32),
        grid=(rows // blk,),
        in_specs=[pl.BlockSpec((blk, _D), lambda i: (i, 0))],
        out_specs=pl.BlockSpec((blk, _D), lambda i: (i, 0)),
    )(mem2)


def _proj_body(m_ref, w_ref, o_ref):
    o_ref[...] = jnp.dot(m_ref[...], w_ref[...],
                         preferred_element_type=jnp.float32)


def _tc_project(msgs, w):
    return pl.pallas_call(
        _proj_body,
        out_shape=jax.ShapeDtypeStruct((_B, _D), jnp.float32),
    )(msgs, w)


_BB = 256


def _compute_body(nc_ref, s_ref, ts_ref, ets_ref, p_ref, cv_ref, hv_ref):
    nc = nc_ref[...]
    s = s_ref[...]
    logits = jnp.sum(nc * s, axis=2, keepdims=True)
    m = jnp.max(logits, axis=1, keepdims=True)
    e = jnp.exp(logits - m)
    att = e / jnp.sum(e, axis=1, keepdims=True)
    delta = ts_ref[...] - ets_ref[...]
    ok = jnp.logical_and(delta > 0.0, delta < _TAU)
    coef = jnp.where(ok, jnp.exp(-_ALPHA * delta) * att, 0.0)
    cv = nc + coef * p_ref[...]
    cv_ref[...] = cv
    hv_ref[...] = jnp.tanh(cv)


def _tc_compute(nc3, s3, ts3, ets3, p_rep):
    grid = _B // _BB
    sds = jax.ShapeDtypeStruct((_B, _NN, _D), jnp.float32)
    return pl.pallas_call(
        _compute_body,
        out_shape=(sds, sds),
        grid=(grid,),
        in_specs=[
            pl.BlockSpec((_BB, _NN, _D), lambda k: (k, 0, 0)),
            pl.BlockSpec((_BB, 1, _D), lambda k: (k, 0, 0)),
            pl.BlockSpec((_BB, 1, 1), lambda k: (k, 0, 0)),
            pl.BlockSpec((_BB, _NN, 1), lambda k: (k, 0, 0)),
            pl.BlockSpec((_BB, _NN, _D), lambda k: (lax.rem(k, 4), 0, 0)),
        ],
        out_specs=(
            pl.BlockSpec((_BB, _NN, _D), lambda k: (k, 0, 0)),
            pl.BlockSpec((_BB, _NN, _D), lambda k: (k, 0, 0)),
        ),
    )(nc3, s3, ts3, ets3, p_rep)


def kernel(memory, unique_node_ids, unique_messages, timestamps,
           neighbors, edge_times, W_s):
    mem2 = memory.reshape(2 * _M, _D)
    flat = neighbors.reshape(-1).astype(jnp.int32)
    flat1 = flat + jnp.int32(_M)
    uids = unique_node_ids.astype(jnp.int32)

    p = _tc_project(unique_messages, W_s)
    p_rep = jnp.tile(p, (5, 1)).reshape(_B // 4, _NN, _D)

    nc_flat, s_flat = _sc_gather(mem2, flat, uids)
    nc3 = nc_flat.reshape(_B, _NN, _D)
    s3 = s_flat.reshape(_B, 1, _D)
    ts3 = timestamps.reshape(_B, 1, 1)
    ets3 = edge_times.reshape(_B, _NN, 1)

    cv3, hv3 = _tc_compute(nc3, s3, ts3, ets3, p_rep)
    cv = cv3.reshape(_B * _NN, _D)
    hv = hv3.reshape(_B * _NN, _D)

    out_ref = jax.new_ref(mem2)
    _sc_scatter(out_ref, cv, hv, flat, flat1)
    return jax.freeze(out_ref).reshape(2, _M, _D)


# speedup vs baseline: 2.1316x; 1.1191x over previous
import functools

import jax
import jax.numpy as jnp
from jax import lax
from jax.experimental import pallas as pl
from jax.experimental.pallas import tpu as pltpu
from jax.experimental.pallas import tpu_sc as plsc

_M = 100000
_D = 64
_B = 4096
_NN = 20
_ALPHA = 1.0 / 100.0
_TAU = 200.0

_NCORE = 2
_NSUB = 16
_NW = _NCORE * _NSUB
_JPW = (_B * _NN) // _NW
_SPW = _B // _NW
_GC = 128
_NCH = _JPW // _GC

_mesh = plsc.VectorSubcoreMesh(core_axis_name="c", subcore_axis_name="s",
                               num_cores=_NCORE, num_subcores=_NSUB)

_sc_params = pltpu.CompilerParams(use_tc_tiling_on_sc=False)

_sc_scratch = [
    pltpu.VMEM((_GC,), jnp.int32),
    pltpu.VMEM((_GC,), jnp.int32),
    pltpu.VMEM((_GC, _D), jnp.float32),
    pltpu.VMEM((_GC, _D), jnp.float32),
] + [pltpu.SemaphoreType.DMA] * 6


def _worker_id():
    return lax.axis_index("c") * _NSUB + lax.axis_index("s")


@functools.partial(
    pl.kernel,
    out_type=(
        jax.ShapeDtypeStruct((_B * _NN, _D), jnp.float32),
        jax.ShapeDtypeStruct((_B, _D), jnp.float32),
    ),
    mesh=_mesh,
    scratch_types=_sc_scratch,
    compiler_params=_sc_params,
)
def _sc_gather(mem_hbm, flat_hbm, uid_hbm, nc_hbm, s_hbm,
               i0, i1, r0, r1, si0, si1, sg0, sg1, ss0, ss1):
    wid = _worker_id()
    jb = wid * _JPW
    sb = wid * _SPW
    ibufs, rbufs = (i0, i1), (r0, r1)
    isems, gsems, ssems = (si0, si1), (sg0, sg1), (ss0, ss1)
    cell_hbm = mem_hbm.at[0]

    pltpu.sync_copy(uid_hbm.at[pl.ds(sb, _SPW)], i0)
    pltpu.async_copy(cell_hbm.at[i0], r0, sg0).wait()
    pltpu.sync_copy(r0, s_hbm.at[pl.ds(sb, _SPW)])

    ih = [None] * _NCH
    gh = [None] * _NCH
    sh = [None] * _NCH

    def issue_i(c):
        b = c % 2
        ih[c] = pltpu.async_copy(
            flat_hbm.at[pl.ds(jb + c * _GC, _GC)], ibufs[b], isems[b])

    def issue_g(c):
        b = c % 2
        gh[c] = pltpu.async_copy(cell_hbm.at[ibufs[b]], rbufs[b], gsems[b])

    def issue_s(c):
        b = c % 2
        sh[c] = pltpu.async_copy(
            rbufs[b], nc_hbm.at[pl.ds(jb + c * _GC, _GC)], ssems[b])

    issue_i(0)
    issue_i(1)
    ih[0].wait()
    issue_g(0)
    ih[1].wait()
    issue_g(1)
    for c in range(_NCH):
        gh[c].wait()
        issue_s(c)
        sh[c].wait()
        if c + 2 < _NCH:
            issue_i(c + 2)
            ih[c + 2].wait()
            issue_g(c + 2)


@functools.partial(
    pl.kernel,
    out_type=(),
    mesh=_mesh,
    scratch_types=_sc_scratch,
    compiler_params=_sc_params,
)
def _sc_scatter(o_hbm, cv_hbm, hv_hbm, f_hbm,
                i0, i1, r0, r1, si0, si1, sr0, sr1, sw0, sw1):
    wid = _worker_id()
    jb = wid * _JPW
    ibufs, rbufs = (i0, i1), (r0, r1)
    isems, rsems, wsems = (si0, si1), (sr0, sr1), (sw0, sw1)

    for src_hbm, plane in ((cv_hbm, 0), (hv_hbm, 1)):
        dst_hbm = o_hbm.at[plane]
        ih = [None] * _NCH
        rh = [None] * _NCH
        wh = [None] * _NCH

        def issue_load(c, src_hbm=src_hbm, ih=ih, rh=rh):
            b = c % 2
            ih[c] = pltpu.async_copy(
                f_hbm.at[pl.ds(jb + c * _GC, _GC)], ibufs[b], isems[b])
            rh[c] = pltpu.async_copy(
                src_hbm.at[pl.ds(jb + c * _GC, _GC)], rbufs[b], rsems[b])

        issue_load(0)
        issue_load(1)
        for c in range(_NCH):
            b = c % 2
            ih[c].wait()
            rh[c].wait()
            wh[c] = pltpu.async_copy(rbufs[b], dst_hbm.at[ibufs[b]], wsems[b])
            wh[c].wait()
            if c + 2 < _NCH:
                issue_load(c + 2)


def _proj_body(m_ref, w_ref, o_ref):
    o_ref[...] = jnp.dot(m_ref[...], w_ref[...],
                         preferred_element_type=jnp.float32)


def _tc_project(msgs, w):
    return pl.pallas_call(
        _proj_body,
        out_shape=jax.ShapeDtypeStruct((5 * _B, _D), jnp.float32),
        grid=(5,),
        in_specs=[
            pl.BlockSpec((_B, _D), lambda i: (0, 0)),
            pl.BlockSpec((_D, _D), lambda i: (0, 0)),
        ],
        out_specs=pl.BlockSpec((_B, _D), lambda i: (i, 0)),
    )(msgs, w)


_BB = 256
_BR = _BB * _NN


def _compute_body(nc_ref, s_ref, ts_ref, ets_ref, p_ref, cv_ref, hv_ref):
    nc2 = nc_ref[...]
    nc3 = nc2.reshape(_BB, _NN, _D)
    s = s_ref[...]
    logits = jnp.sum(nc3 * s[:, None, :], axis=2)
    m = jnp.max(logits, axis=1, keepdims=True)
    e = jnp.exp(logits - m)
    att = e / jnp.sum(e, axis=1, keepdims=True)
    delta = ts_ref[...] - ets_ref[...]
    ok = jnp.logical_and(delta > 0.0, delta < _TAU)
    coef = jnp.where(ok, jnp.exp(-_ALPHA * delta) * att, 0.0)
    p3 = p_ref[...].reshape(_BB, _NN, _D)
    cv3 = nc3 + coef[:, :, None] * p3
    cv = cv3.reshape(_BR, _D)
    cv_ref[...] = cv
    hv_ref[...] = jnp.tanh(cv)


def _tc_compute(nc, s, ts2, ets, p5):
    grid = _B // _BB
    sds = jax.ShapeDtypeStruct((_B * _NN, _D), jnp.float32)
    return pl.pallas_call(
        _compute_body,
        out_shape=(sds, sds),
        grid=(grid,),
        in_specs=[
            pl.BlockSpec((_BR, _D), lambda k: (k, 0)),
            pl.BlockSpec((_BB, _D), lambda k: (k, 0)),
            pl.BlockSpec((_BB, 1), lambda k: (k, 0)),
            pl.BlockSpec((_BB, _NN), lambda k: (k, 0)),
            pl.BlockSpec((_BR, _D), lambda k: (lax.rem(k, 4), 0)),
        ],
        out_specs=(
            pl.BlockSpec((_BR, _D), lambda k: (k, 0)),
            pl.BlockSpec((_BR, _D), lambda k: (k, 0)),
        ),
    )(nc, s, ts2, ets, p5)


def kernel(memory, unique_node_ids, unique_messages, timestamps,
           neighbors, edge_times, W_s):
    flat = neighbors.reshape(-1).astype(jnp.int32)
    uids = unique_node_ids.astype(jnp.int32)

    p5 = _tc_project(unique_messages, W_s)
    nc_flat, s_flat = _sc_gather(memory, flat, uids)
    ts2 = timestamps.reshape(_B, 1)

    cv, hv = _tc_compute(nc_flat, s_flat, ts2, edge_times, p5)

    out_ref = jax.new_ref(memory)
    _sc_scatter(out_ref, cv, hv, flat)
    return jax.freeze(out_ref)
